# Initial kernel scaffold; baseline (speedup 1.0000x reference)
import jax
import jax.numpy as jnp
from jax.experimental import pallas as pl


def kernel(features, edge_index, pseudo, weight, bias):
    raise NotImplementedError("write your pallas kernel here")


# speedup vs baseline: 1.3982x; 1.3982x over previous
import functools
import math

import jax
import jax.numpy as jnp
from jax import lax
from jax.experimental import pallas as pl
from jax.experimental.pallas import tpu as pltpu
from jax.experimental.pallas import tpu_sc as plsc

N = 10000
E = 320000
IN = 128
OUT = 128
K1 = 5
K2 = 5
K = K1 * K2

NC = 2
NS = 16
NW = NC * NS
B = 80
L = 16


def _mm_body(x_ref, w_ref, o_ref):
    o_ref[...] = jnp.dot(x_ref[...], w_ref[...],
                         preferred_element_type=jnp.float32)


def _node_transform(features, w2):
    blk = 1000
    grid = (N // blk,)
    return pl.pallas_call(
        _mm_body,
        grid=grid,
        in_specs=[
            pl.BlockSpec((blk, IN), lambda i: (i, 0)),
            pl.BlockSpec((IN, K * OUT), lambda i: (0, 0)),
        ],
        out_specs=pl.BlockSpec((blk, K * OUT), lambda i: (i, 0)),
        out_shape=jax.ShapeDtypeStruct((N, K * OUT), jnp.float32),
    )(features, w2)


def _sc_body(table, srcv, dstv, uu, vv, partial,
             sb, db, ub, vb, idx4, coef, rows, sem):
    c = lax.axis_index("c")
    s = lax.axis_index("s")
    wid = s * NC + c
    ep = E // NW
    nblk = ep // B
    zbuf = rows.at[0]

    def _zero_row(i, _):
        for j in range(OUT // L):
            rows[0, i, pl.ds(j * L, L)] = jnp.zeros((L,), jnp.float32)
        return 0
    lax.fori_loop(0, B, _zero_row, 0)
    row0 = s * 640

    @pl.when(s < NS - 1)
    def _zero_full():
        for t in range(640 // B):
            pltpu.sync_copy(zbuf, partial.at[pl.ds(row0 + t * B, B)])

    @pl.when(s == NS - 1)
    def _zero_last():
        for t in range((N - 640 * (NS - 1)) // B):
            pltpu.sync_copy(zbuf, partial.at[pl.ds(row0 + t * B, B)])

    plsc.subcore_barrier()

    def _block(t, _):
        base = wid * ep + t * B
        pltpu.sync_copy(srcv.at[pl.ds(base, B)], sb)
        pltpu.sync_copy(dstv.at[pl.ds(base, B)], db)
        pltpu.sync_copy(uu.at[pl.ds(base, B)], ub)
        pltpu.sync_copy(vv.at[pl.ds(base, B)], vb)

        for j in range(B // L):
            sl = pl.ds(j * L, L)
            p0 = ub[sl] * float(K1 - 1)
            p1 = vb[sl] * float(K2 - 1)
            b0 = jnp.clip(p0.astype(jnp.int32), 0, K1 - 2)
            b1 = jnp.clip(p1.astype(jnp.int32), 0, K2 - 2)
            f0 = p0 - b0.astype(jnp.float32)
            f1 = p1 - b1.astype(jnp.float32)
            r00 = sb[sl] * K + b0 * K2 + b1
            idx4[0, sl] = r00
            idx4[1, sl] = r00 + 1
            idx4[2, sl] = r00 + K2
            idx4[3, sl] = r00 + K2 + 1
            g0 = 1.0 - f0
            g1 = 1.0 - f1
            coef[j, 0] = g0 * g1
            coef[j, 1] = g0 * f1
            coef[j, 2] = f0 * g1
            coef[j, 3] = f0 * f1

        cps = [pltpu.async_copy(table.at[idx4.at[q]], rows.at[q], sem)
               for q in range(4)]
        for cp in cps:
            cp.wait()

        def _group(g, _):
            c0v = coef[g, 0]
            c1v = coef[g, 1]
            c2v = coef[g, 2]
            c3v = coef[g, 3]
            for l in range(L):
                i = g * L + l
                c0 = c0v[l]
                c1 = c1v[l]
                c2 = c2v[l]
                c3 = c3v[l]
                for j in range(OUT // L):
                    sl = pl.ds(j * L, L)
                    m = (c0 * rows[0, i, sl] + c1 * rows[1, i, sl]
                         + c2 * rows[2, i, sl] + c3 * rows[3, i, sl])
                    rows[0, i, sl] = m
            return 0
        lax.fori_loop(0, B // L, _group, 0)

        pltpu.sync_copy(rows.at[0], partial.at[db], add=True)
        return 0

    lax.fori_loop(0, nblk, _block, 0)
    plsc.subcore_barrier()

    pass


def _sc_edges(table, srcv, dstv, uu, vv):
    mesh = plsc.VectorSubcoreMesh(core_axis_name="c", subcore_axis_name="s")

    @functools.partial(
        pl.kernel,
        out_type=jax.ShapeDtypeStruct((NC, N, OUT), jnp.float32),
        mesh=mesh,
        scratch_types=[
            pltpu.VMEM((B,), jnp.int32),
            pltpu.VMEM((B,), jnp.int32),
            pltpu.VMEM((B,), jnp.float32),
            pltpu.VMEM((B,), jnp.float32),
            pltpu.VMEM((4, B), jnp.int32),
            pltpu.VMEM((B // L, 4, L), jnp.float32),
            pltpu.VMEM((4, B, OUT), jnp.float32),
            pltpu.VMEM_SHARED((N, OUT), jnp.float32),
            pltpu.SemaphoreType.DMA,
        ],
    )
    def k(table_h, src_h, dst_h, u_h, v_h, out_h,
          sb, db, ub, vb, idx4, coef, rows, shared, sem):
        _sc_body(table_h, src_h, dst_h, u_h, v_h, shared,
                 sb, db, ub, vb, idx4, coef, rows, sem)
        c = lax.axis_index("c")
        s = lax.axis_index("s")
        row0 = s * 640

        bounce = rows.at[0]

        @pl.when(s < NS - 1)
        def _copy_full():
            for t in range(640 // B):
                pltpu.sync_copy(shared.at[pl.ds(row0 + t * B, B)], bounce)
                pltpu.sync_copy(bounce, out_h.at[c, pl.ds(row0 + t * B, B)])

        @pl.when(s == NS - 1)
        def _copy_last():
            for t in range((N - 640 * (NS - 1)) // B):
                pltpu.sync_copy(shared.at[pl.ds(row0 + t * B, B)], bounce)
                pltpu.sync_copy(bounce, out_h.at[c, pl.ds(row0 + t * B, B)])

    return k(table, srcv, dstv, uu, vv)


def _comb_body(p_ref, b_ref, o_ref):
    o_ref[...] = p_ref[0] + p_ref[1] + b_ref[...]


def _combine(partial, bias2):
    blk = 1000
    return pl.pallas_call(
        _comb_body,
        grid=(N // blk,),
        in_specs=[
            pl.BlockSpec((NC, blk, OUT), lambda i: (0, i, 0)),
            pl.BlockSpec((1, OUT), lambda i: (0, 0)),
        ],
        out_specs=pl.BlockSpec((blk, OUT), lambda i: (i, 0)),
        out_shape=jax.ShapeDtypeStruct((N, OUT), jnp.float32),
    )(partial, bias2)


def kernel(features, edge_index, pseudo, weight, bias):
    w2 = weight.reshape(K, IN, OUT).transpose(1, 0, 2).reshape(IN, K * OUT)
    table = _node_transform(features, w2).reshape(N * K, OUT)
    srcv = edge_index[0]
    dstv = edge_index[1]
    uu = pseudo[:, 0]
    vv = pseudo[:, 1]
    partial = _sc_edges(table, srcv, dstv, uu, vv)
    return _combine(partial, bias.reshape(1, OUT))


# speedup vs baseline: 1.7697x; 1.2657x over previous
import functools

import jax
import jax.numpy as jnp
from jax import lax
from jax.experimental import pallas as pl
from jax.experimental.pallas import tpu as pltpu
from jax.experimental.pallas import tpu_sc as plsc

N = 10000
E = 320000
IN = 128
OUT = 128
K1 = 5
K2 = 5
K = K1 * K2

NC = 2
NS = 16
NW = NC * NS
B = 80
L = 16


def _rne16(r):
    bits = lax.bitcast_convert_type(r, jnp.int32)
    rnd = jnp.right_shift(bits, 16) & 1
    return jnp.right_shift(bits + 32767 + rnd, 16)


def _mm_body(x_ref, we_ref, wo_ref, o_ref):
    x = x_ref[...]
    re = jnp.dot(x, we_ref[...], preferred_element_type=jnp.float32)
    ro = jnp.dot(x, wo_ref[...], preferred_element_type=jnp.float32)
    o_ref[...] = (_rne16(re) & 0xFFFF) | jnp.left_shift(_rne16(ro), 16)


NP = K - 1


def _node_transform(features, we, wo):
    blk = 1000
    grid = (N // blk,)
    return pl.pallas_call(
        _mm_body,
        grid=grid,
        in_specs=[
            pl.BlockSpec((blk, IN), lambda i: (i, 0)),
            pl.BlockSpec((IN, NP * OUT), lambda i: (0, 0)),
            pl.BlockSpec((IN, NP * OUT), lambda i: (0, 0)),
        ],
        out_specs=pl.BlockSpec((blk, NP * OUT), lambda i: (i, 0)),
        out_shape=jax.ShapeDtypeStruct((N, NP * OUT), jnp.int32),
    )(features, we, wo)


def _sc_body(table, meta, partial, mb, db, idx4, coef, rows, msg, sem):
    c = lax.axis_index("c")
    s = lax.axis_index("s")
    wid = s * NC + c
    ep = E // NW
    nblk = ep // B

    def _zero_row(i, _):
        for j in range(OUT // L):
            msg[i, pl.ds(j * L, L)] = jnp.zeros((L,), jnp.float32)
        return 0
    lax.fori_loop(0, B, _zero_row, 0)
    row0 = s * 640

    @pl.when(s < NS - 1)
    def _zero_full():
        for t in range(640 // B):
            pltpu.sync_copy(msg, partial.at[pl.ds(row0 + t * B, B)])

    @pl.when(s == NS - 1)
    def _zero_last():
        for t in range((N - 640 * (NS - 1)) // B):
            pltpu.sync_copy(msg, partial.at[pl.ds(row0 + t * B, B)])

    plsc.subcore_barrier()

    def _block(t, _):
        base = wid * ep + t * B
        pltpu.sync_copy(meta.at[pl.ds(base // L, B // L)], mb)

        for j in range(B // L):
            sl = pl.ds(j * L, L)
            u = lax.bitcast_convert_type(mb[j, 2], jnp.float32)
            v = lax.bitcast_convert_type(mb[j, 3], jnp.float32)
            db[sl] = mb[j, 1]
            p0 = u * float(K1 - 1)
            p1 = v * float(K2 - 1)
            b0 = jnp.clip(p0.astype(jnp.int32), 0, K1 - 2)
            b1 = jnp.clip(p1.astype(jnp.int32), 0, K2 - 2)
            f0 = p0 - b0.astype(jnp.float32)
            f1 = p1 - b1.astype(jnp.float32)
            r00 = mb[j, 0] * (K - 1) + b0 * K2 + b1
            idx4[0, sl] = r00
            idx4[1, sl] = r00 + K2
            g0 = 1.0 - f0
            g1 = 1.0 - f1
            coef[j, 0] = g0 * g1
            coef[j, 1] = g0 * f1
            coef[j, 2] = f0 * g1
            coef[j, 3] = f0 * f1

        cps = [pltpu.async_copy(table.at[idx4.at[q]], rows.at[q], sem)
               for q in range(2)]
        for cp in cps:
            cp.wait()

        def _group(g, _):
            c0v = coef[g, 0]
            c1v = coef[g, 1]
            c2v = coef[g, 2]
            c3v = coef[g, 3]
            for l in range(L):
                i = g * L + l
                c0 = c0v[l]
                c1 = c1v[l]
                c2 = c2v[l]
                c3 = c3v[l]
                for j in range(OUT // 32):
                    hm = jnp.int32(-65536)
                    w0 = rows[0, i, pl.ds(j * L, L)]
                    w1 = rows[0, i, pl.ds(64 + j * L, L)]
                    w2_ = rows[1, i, pl.ds(j * L, L)]
                    w3 = rows[1, i, pl.ds(64 + j * L, L)]
                    r0lo = lax.bitcast_convert_type(w0 << 16, jnp.float32)
                    r1lo = lax.bitcast_convert_type(w1 << 16, jnp.float32)
                    r2lo = lax.bitcast_convert_type(w2_ << 16, jnp.float32)
                    r3lo = lax.bitcast_convert_type(w3 << 16, jnp.float32)
                    r0hi = lax.bitcast_convert_type(w0 & hm, jnp.float32)
                    r1hi = lax.bitcast_convert_type(w1 & hm, jnp.float32)
                    r2hi = lax.bitcast_convert_type(w2_ & hm, jnp.float32)
                    r3hi = lax.bitcast_convert_type(w3 & hm, jnp.float32)
                    mlo = c0 * r0lo + c1 * r1lo + c2 * r2lo + c3 * r3lo
                    mhi = c0 * r0hi + c1 * r1hi + c2 * r2hi + c3 * r3hi
                    msg[i, pl.ds(j * 32, L)] = mlo
                    msg[i, pl.ds(j * 32 + L, L)] = mhi
            return 0
        lax.fori_loop(0, B // L, _group, 0)

        pltpu.sync_copy(msg, partial.at[db], add=True)
        return 0

    lax.fori_loop(0, nblk, _block, 0)
    plsc.subcore_barrier()


def _sc_edges(table, meta):
    mesh = plsc.VectorSubcoreMesh(core_axis_name="c", subcore_axis_name="s")

    @functools.partial(
        pl.kernel,
        out_type=jax.ShapeDtypeStruct((NC, N, OUT), jnp.float32),
        mesh=mesh,
        scratch_types=[
            pltpu.VMEM((B // L, 4, L), jnp.int32),
            pltpu.VMEM((B,), jnp.int32),
            pltpu.VMEM((2, B), jnp.int32),
            pltpu.VMEM((B // L, 4, L), jnp.float32),
            pltpu.VMEM((2, B, OUT), jnp.int32),
            pltpu.VMEM((B, OUT), jnp.float32),
            pltpu.VMEM_SHARED((N, OUT), jnp.float32),
            pltpu.SemaphoreType.DMA,
        ],
    )
    def k(table_h, meta_h, out_h, mb, db, idx4, coef, rows, msg, shared, sem):
        _sc_body(table_h, meta_h, shared, mb, db, idx4, coef, rows, msg, sem)
        c = lax.axis_index("c")
        s = lax.axis_index("s")
        row0 = s * 640

        @pl.when(s < NS - 1)
        def _copy_full():
            for t in range(640 // B):
                pltpu.sync_copy(shared.at[pl.ds(row0 + t * B, B)], msg)
                pltpu.sync_copy(msg, out_h.at[c, pl.ds(row0 + t * B, B)])

        @pl.when(s == NS - 1)
        def _copy_last():
            for t in range((N - 640 * (NS - 1)) // B):
                pltpu.sync_copy(shared.at[pl.ds(row0 + t * B, B)], msg)
                pltpu.sync_copy(msg, out_h.at[c, pl.ds(row0 + t * B, B)])

    return k(table, meta)


def _comb_body(p_ref, b_ref, o_ref):
    o_ref[...] = p_ref[0] + p_ref[1] + b_ref[...]


def _combine(partial, bias2):
    blk = 1000
    return pl.pallas_call(
        _comb_body,
        grid=(N // blk,),
        in_specs=[
            pl.BlockSpec((NC, blk, OUT), lambda i: (0, i, 0)),
            pl.BlockSpec((1, OUT), lambda i: (0, 0)),
        ],
        out_specs=pl.BlockSpec((blk, OUT), lambda i: (i, 0)),
        out_shape=jax.ShapeDtypeStruct((N, OUT), jnp.float32),
    )(partial, bias2)


def kernel(features, edge_index, pseudo, weight, bias):
    ch = jnp.arange(OUT).reshape(-1, 2, L)
    wk = weight.reshape(K, IN, OUT)
    we_k = wk[:, :, ch[:, 0].reshape(-1)]
    wo_k = wk[:, :, ch[:, 1].reshape(-1)]
    kf = jnp.arange(K - 1)
    pair = jnp.stack([kf, kf + 1], axis=1).reshape(-1)
    we = we_k[pair].transpose(1, 0, 2).reshape(IN, (K - 1) * OUT)
    wo = wo_k[pair].transpose(1, 0, 2).reshape(IN, (K - 1) * OUT)
    table = _node_transform(features, we, wo).reshape(N * (K - 1), OUT)
    meta = jnp.stack([
        edge_index[0],
        edge_index[1],
        lax.bitcast_convert_type(pseudo[:, 0], jnp.int32),
        lax.bitcast_convert_type(pseudo[:, 1], jnp.int32),
    ]).reshape(4, E // L, L).transpose(1, 0, 2)
    partial = _sc_edges(table, meta)
    return _combine(partial, bias.reshape(1, OUT))


# speedup vs baseline: 2.1222x; 1.1992x over previous
import functools

import jax
import jax.numpy as jnp
from jax import lax
from jax.experimental import pallas as pl
from jax.experimental.pallas import tpu as pltpu
from jax.experimental.pallas import tpu_sc as plsc

N = 10000
E = 320000
IN = 128
OUT = 128
K1 = 5
K2 = 5
K = K1 * K2
NP = K - 1

NC = 2
NS = 16
NW = NC * NS
B = 64
L = 16
NBLK = -(-E // (NW * B))
if NBLK % 2 == 0:
    NBLK += 1
EPAD = NBLK * NW * B
NPAD = N + 8
HM = jnp.int32(-65536)


def _rne16(r):
    bits = lax.bitcast_convert_type(r, jnp.int32)
    rnd = jnp.right_shift(bits, 16) & 1
    return jnp.right_shift(bits + 32767 + rnd, 16)


def _mm_body(x_ref, we_ref, wo_ref, o_ref):
    x = x_ref[...]
    re = jnp.dot(x, we_ref[...], preferred_element_type=jnp.float32)
    ro = jnp.dot(x, wo_ref[...], preferred_element_type=jnp.float32)
    packed = (_rne16(re) & 0xFFFF) | jnp.left_shift(_rne16(ro), 16)
    o_ref[...] = lax.bitcast_convert_type(packed, jnp.float32)


def _node_transform(features, we, wo):
    blk = 1000
    grid = (N // blk,)
    return pl.pallas_call(
        _mm_body,
        grid=grid,
        in_specs=[
            pl.BlockSpec((blk, IN), lambda i: (i, 0)),
            pl.BlockSpec((IN, NP * OUT), lambda i: (0, 0)),
            pl.BlockSpec((IN, NP * OUT), lambda i: (0, 0)),
        ],
        out_specs=pl.BlockSpec((blk, NP * OUT), lambda i: (i, 0)),
        out_shape=jax.ShapeDtypeStruct((N, NP * OUT), jnp.float32),
    )(features, we, wo)


def _bc_i32(x):
    return lax.bitcast_convert_type(x, jnp.int32)


def _bc_f32(x):
    return lax.bitcast_convert_type(x, jnp.float32)


def _sc_body(table, meta, partial, mb, db, idx2, coef, rows, sems):
    c = lax.axis_index("c")
    s = lax.axis_index("s")
    wid = s * NC + c
    ep = EPAD // NW
    nblk = ep // B

    def _zero_row(i, _):
        for j in range(OUT // L):
            rows[0, i, pl.ds(j * L, L)] = jnp.zeros((L,), jnp.float32)
        return 0
    lax.fori_loop(0, B, _zero_row, 0)
    row0 = s * 640

    @pl.when(s < NS - 1)
    def _zero_full():
        for t in range(640 // B):
            pltpu.sync_copy(rows.at[0], partial.at[pl.ds(row0 + t * B, B)])

    @pl.when(s == NS - 1)
    def _zero_last():
        last = 640 * (NS - 1)
        for t in range((NPAD - last) // B):
            pltpu.sync_copy(rows.at[0], partial.at[pl.ds(last + t * B, B)])
        if (NPAD - last) % B:
            pltpu.sync_copy(rows.at[0], partial.at[pl.ds(NPAD - B, B)])

    plsc.subcore_barrier()

    def _prep(t, slot):
        base = wid * ep + t * B
        pltpu.sync_copy(meta.at[pl.ds(base // L, B // L)], mb)
        for j in range(B // L):
            sl = pl.ds(j * L, L)
            u = _bc_f32(mb[j, 2])
            v = _bc_f32(mb[j, 3])
            db[slot, sl] = mb[j, 1]
            p0 = u * float(K1 - 1)
            p1 = v * float(K2 - 1)
            b0 = jnp.clip(p0.astype(jnp.int32), 0, K1 - 2)
            b1 = jnp.clip(p1.astype(jnp.int32), 0, K2 - 2)
            f0 = p0 - b0.astype(jnp.float32)
            f1 = p1 - b1.astype(jnp.float32)
            r00 = mb[j, 0] * NP + b0 * K2 + b1
            idx2[slot, 0, sl] = r00
            idx2[slot, 1, sl] = r00 + K2
            g0 = 1.0 - f0
            g1 = 1.0 - f1
            coef[slot, j, 0] = g0 * g1
            coef[slot, j, 1] = g0 * f1
            coef[slot, j, 2] = f0 * g1
            coef[slot, j, 3] = f0 * f1
        for q in range(2):
            pltpu.async_copy(table.at[idx2.at[slot, q]],
                             rows.at[2 * slot + q], sems[slot])

    def _wait(slot):
        for q in range(2):
            pltpu.make_async_copy(table.at[idx2.at[slot, q]],
                                  rows.at[2 * slot + q], sems[slot]).wait()

    def _combine_scatter(slot):
        p0 = 2 * slot
        p1 = 2 * slot + 1

        def _group(g, _):
            c0v = coef[slot, g, 0]
            c1v = coef[slot, g, 1]
            c2v = coef[slot, g, 2]
            c3v = coef[slot, g, 3]
            for l in range(L):
                i = g * L + l
                c0 = c0v[l]
                c1 = c1v[l]
                c2 = c2v[l]
                c3 = c3v[l]
                a = [_bc_i32(rows[p0, i, pl.ds(jj * L, L)]) for jj in range(8)]
                for j in range(4):
                    w0 = a[j]
                    w1 = a[4 + j]
                    w2_ = _bc_i32(rows[p1, i, pl.ds(j * L, L)])
                    w3 = _bc_i32(rows[p1, i, pl.ds(64 + j * L, L)])
                    mlo = (c0 * _bc_f32(w0 << 16) + c1 * _bc_f32(w1 << 16)
                           + c2 * _bc_f32(w2_ << 16) + c3 * _bc_f32(w3 << 16))
                    mhi = (c0 * _bc_f32(w0 & HM) + c1 * _bc_f32(w1 & HM)
                           + c2 * _bc_f32(w2_ & HM) + c3 * _bc_f32(w3 & HM))
                    rows[p0, i, pl.ds(j * 32, L)] = mlo
                    rows[p0, i, pl.ds(j * 32 + L, L)] = mhi
            return 0
        lax.fori_loop(0, B // L, _group, 0)
        pltpu.sync_copy(rows.at[p0], partial.at[db.at[slot]], add=True)

    _prep(0, 0)

    def _pair(t2, _):
        t0 = 2 * t2
        _prep(t0 + 1, 1)
        _wait(0)
        _combine_scatter(0)
        _prep(t0 + 2, 0)
        _wait(1)
        _combine_scatter(1)
        return 0
    lax.fori_loop(0, (nblk - 1) // 2, _pair, 0)
    _wait(0)
    _combine_scatter(0)

    plsc.subcore_barrier()


def _sc_edges(table, meta):
    mesh = plsc.VectorSubcoreMesh(core_axis_name="c", subcore_axis_name="s")

    @functools.partial(
        pl.kernel,
        out_type=jax.ShapeDtypeStruct((NC, N, OUT), jnp.float32),
        mesh=mesh,
        scratch_types=[
            pltpu.VMEM((B // L, 4, L), jnp.int32),
            pltpu.VMEM((2, B), jnp.int32),
            pltpu.VMEM((2, 2, B), jnp.int32),
            pltpu.VMEM((2, B // L, 4, L), jnp.float32),
            pltpu.VMEM((4, B, OUT), jnp.float32),
            pltpu.VMEM_SHARED((NPAD, OUT), jnp.float32),
            pltpu.SemaphoreType.DMA,
            pltpu.SemaphoreType.DMA,
        ],
    )
    def k(table_h, meta_h, out_h, mb, db, idx2, coef, rows, shared,
          sem0, sem1):
        _sc_body(table_h, meta_h, shared, mb, db, idx2, coef, rows,
                 (sem0, sem1))
        c = lax.axis_index("c")
        s = lax.axis_index("s")
        row0 = s * 640
        bounce = rows.at[0]

        @pl.when(s < NS - 1)
        def _copy_full():
            for t in range(640 // B):
                pltpu.sync_copy(shared.at[pl.ds(row0 + t * B, B)], bounce)
                pltpu.sync_copy(bounce, out_h.at[c, pl.ds(row0 + t * B, B)])

        @pl.when(s == NS - 1)
        def _copy_last():
            last = 640 * (NS - 1)
            for t in range((N - last) // B):
                pltpu.sync_copy(shared.at[pl.ds(last + t * B, B)], bounce)
                pltpu.sync_copy(bounce, out_h.at[c, pl.ds(last + t * B, B)])
            if (N - last) % B:
                pltpu.sync_copy(shared.at[pl.ds(N - B, B)], bounce)
                pltpu.sync_copy(bounce, out_h.at[c, pl.ds(N - B, B)])

    return k(table, meta)


def _comb_body(p_ref, b_ref, o_ref):
    o_ref[...] = p_ref[0] + p_ref[1] + b_ref[...]


def _combine(partial, bias2):
    blk = 1000
    return pl.pallas_call(
        _comb_body,
        grid=(N // blk,),
        in_specs=[
            pl.BlockSpec((NC, blk, OUT), lambda i: (0, i, 0)),
            pl.BlockSpec((1, OUT), lambda i: (0, 0)),
        ],
        out_specs=pl.BlockSpec((blk, OUT), lambda i: (i, 0)),
        out_shape=jax.ShapeDtypeStruct((N, OUT), jnp.float32),
    )(partial, bias2)


def kernel(features, edge_index, pseudo, weight, bias):
    ch = jnp.arange(OUT).reshape(-1, 2, L)
    wk = weight.reshape(K, IN, OUT)
    we_k = wk[:, :, ch[:, 0].reshape(-1)]
    wo_k = wk[:, :, ch[:, 1].reshape(-1)]
    kf = jnp.arange(NP)
    pair = jnp.stack([kf, kf + 1], axis=1).reshape(-1)
    we = we_k[pair].transpose(1, 0, 2).reshape(IN, NP * OUT)
    wo = wo_k[pair].transpose(1, 0, 2).reshape(IN, NP * OUT)
    table = _node_transform(features, we, wo).reshape(N * NP, OUT)
    pad = EPAD - E
    meta = jnp.stack([
        jnp.concatenate([edge_index[0], jnp.zeros((pad,), jnp.int32)]),
        jnp.concatenate([edge_index[1], jnp.full((pad,), N, jnp.int32)]),
        jnp.concatenate([lax.bitcast_convert_type(pseudo[:, 0], jnp.int32),
                         jnp.zeros((pad,), jnp.int32)]),
        jnp.concatenate([lax.bitcast_convert_type(pseudo[:, 1], jnp.int32),
                         jnp.zeros((pad,), jnp.int32)]),
    ]).reshape(4, EPAD // L, L).transpose(1, 0, 2)
    partial = _sc_edges(table, meta)
    return _combine(partial, bias.reshape(1, OUT))
